# baseline (speedup 1.0000x reference)
import jax
import jax.numpy as jnp
from jax.experimental import pallas as pl

_B, _L, _C = 4096, 20, 1000
_N = _B * _L
_ROWS = 1024
_H = _N // 2


def _onehot_two(x_ref, x2_ref, o_ref, o2_ref):
    col = jax.lax.broadcasted_iota(jnp.int32, (_ROWS, _C + 1), 1)
    xb = x_ref[...]
    act = xb[:, 0:1].astype(jnp.int32)
    dur = xb[:, 1:2]
    o_ref[...] = (col == act).astype(jnp.float32)
    o_ref[:, _C:_C + 1] = dur
    xb2 = x2_ref[...]
    act2 = xb2[:, 0:1].astype(jnp.int32)
    dur2 = xb2[:, 1:2]
    o2_ref[...] = (col == act2).astype(jnp.float32)
    o2_ref[:, _C:_C + 1] = dur2


def kernel(x):
    xf = x.reshape(_N, 2)
    o1, o2 = pl.pallas_call(
        _onehot_two,
        grid=(_H // _ROWS,),
        in_specs=[pl.BlockSpec((_ROWS, 2), lambda i: (i, 0)),
                  pl.BlockSpec((_ROWS, 2), lambda i: (i, 0))],
        out_specs=[pl.BlockSpec((_ROWS, _C + 1), lambda i: (i, 0)),
                   pl.BlockSpec((_ROWS, _C + 1), lambda i: (i, 0))],
        out_shape=[jax.ShapeDtypeStruct((_H, _C + 1), jnp.float32),
                   jax.ShapeDtypeStruct((_H, _C + 1), jnp.float32)],
    )(xf[:_H], xf[_H:])
    return jnp.concatenate([o1, o2], axis=0).reshape(_B, _L, _C + 1)


# speedup vs baseline: 1.2567x; 1.2567x over previous
import jax
import jax.numpy as jnp
from jax.experimental import pallas as pl

_B, _L, _C = 4096, 20, 1000
_N = _B * _L
_ROWS = 1024
_NSTEP = _N // _ROWS


def _onehot_block(x_ref, o_ref, o2_ref):
    col = jax.lax.broadcasted_iota(jnp.int32, (_ROWS, _C + 1), 1)
    xb = x_ref[...]
    act = xb[:, 0:1].astype(jnp.int32)
    dur = xb[:, 1:2]
    o_ref[...] = (col == act).astype(jnp.float32)
    o_ref[:, _C:_C + 1] = dur
    o2_ref[...] = dur + col[:, 0:128].astype(jnp.float32)


def kernel(x):
    xf = x.reshape(_N, 2)
    out, _ = pl.pallas_call(
        _onehot_block,
        grid=(_NSTEP,),
        in_specs=[pl.BlockSpec((_ROWS, 2), lambda i: (i, 0))],
        out_specs=[pl.BlockSpec((_ROWS, _C + 1), lambda i: (i, 0)),
                   pl.BlockSpec((_ROWS, 128), lambda i: (i, 0))],
        out_shape=[jax.ShapeDtypeStruct((_N, _C + 1), jnp.float32),
                   jax.ShapeDtypeStruct((_N, 128), jnp.float32)],
    )(xf)
    return out.reshape(_B, _L, _C + 1)
